# Initial kernel scaffold; baseline (speedup 1.0000x reference)
import jax
import jax.numpy as jnp
from jax.experimental import pallas as pl
from jax.experimental.pallas import tpu as pltpu

GRID_N = 1024
_H = 1.0 / (GRID_N + 1)
_INV_H2 = 1.0 / (_H * _H)
_MAXITER = 5


def _matvec2d(u):
    n = GRID_N
    zrow = jnp.zeros((1, n), jnp.float32)
    zcol = jnp.zeros((n, 1), jnp.float32)
    up = jnp.concatenate([zrow, u[:-1, :]], axis=0)
    down = jnp.concatenate([u[1:, :], zrow], axis=0)
    left = jnp.concatenate([zcol, u[:, :-1]], axis=1)
    right = jnp.concatenate([u[:, 1:], zcol], axis=1)
    lap = (4.0 * u - up - down - left - right) * jnp.float32(_INV_H2)
    return lap + u


def _descent_kernel(pre_ref, f_ref, out_ref):
    n = GRID_N
    row = jax.lax.broadcasted_iota(jnp.float32, (n, n), 0)
    col = jax.lax.broadcasted_iota(jnp.float32, (n, n), 1)
    xs_r = (row + 1.0) * jnp.float32(_H)
    xs_c = (col + 1.0) * jnp.float32(_H)
    phi = (xs_r * (1.0 - xs_r)) * (xs_c * (1.0 - xs_c))

    u = pre_ref[...] * phi
    b = f_ref[...]
    x = u
    for _ in range(_MAXITER):
        ax = _matvec2d(x)
        r = b - ax
        ar = _matvec2d(r)
        alpha = jnp.sum(r * r) / jnp.sum(r * ar)
        x = x + alpha * r
    diff = u - x
    out_ref[0, 0] = jnp.sum(diff * diff) / jnp.float32(n * n)


def kernel(pre, f, ans):
    del ans
    pre2d = pre.reshape(GRID_N, GRID_N)
    f2d = f.reshape(GRID_N, GRID_N)
    loss = pl.pallas_call(
        _descent_kernel,
        out_shape=jax.ShapeDtypeStruct((1, 1), jnp.float32),
        in_specs=[
            pl.BlockSpec(memory_space=pltpu.VMEM),
            pl.BlockSpec(memory_space=pltpu.VMEM),
        ],
        out_specs=pl.BlockSpec(memory_space=pltpu.SMEM),
    )(pre2d, f2d)
    return loss[0, 0]


# speedup vs baseline: 3.1606x; 3.1606x over previous
import jax
import jax.numpy as jnp
from jax.experimental import pallas as pl
from jax.experimental.pallas import tpu as pltpu

GRID_N = 1024
_H = 1.0 / (GRID_N + 1)
_INV_H2 = 1.0 / (_H * _H)
_MAXITER = 5


def _matvec2d(u):
    n = GRID_N
    zrow = jnp.zeros((1, n), jnp.float32)
    zcol = jnp.zeros((n, 1), jnp.float32)
    up = jnp.concatenate([zrow, u[:-1, :]], axis=0)
    down = jnp.concatenate([u[1:, :], zrow], axis=0)
    left = jnp.concatenate([zcol, u[:, :-1]], axis=1)
    right = jnp.concatenate([u[:, 1:], zcol], axis=1)
    lap = (4.0 * u - up - down - left - right) * jnp.float32(_INV_H2)
    return lap + u


def _descent_kernel(pre_ref, f_ref, out_ref):
    n = GRID_N
    row = jax.lax.broadcasted_iota(jnp.int32, (n, n), 0).astype(jnp.float32)
    col = jax.lax.broadcasted_iota(jnp.int32, (n, n), 1).astype(jnp.float32)
    xs_r = (row + 1.0) * jnp.float32(_H)
    xs_c = (col + 1.0) * jnp.float32(_H)
    phi = (xs_r * (1.0 - xs_r)) * (xs_c * (1.0 - xs_c))

    u = pre_ref[...] * phi
    b = f_ref[...]
    x = u
    for _ in range(_MAXITER):
        ax = _matvec2d(x)
        r = b - ax
        ar = _matvec2d(r)
        alpha = jnp.sum(r * r) / jnp.sum(r * ar)
        x = x + alpha * r
    diff = u - x
    out_ref[0, 0] = jnp.sum(diff * diff) / jnp.float32(n * n)


def kernel(pre, f, ans):
    del ans
    pre2d = pre.reshape(GRID_N, GRID_N)
    f2d = f.reshape(GRID_N, GRID_N)
    loss = pl.pallas_call(
        _descent_kernel,
        out_shape=jax.ShapeDtypeStruct((1, 1), jnp.float32),
        in_specs=[
            pl.BlockSpec(memory_space=pltpu.VMEM),
            pl.BlockSpec(memory_space=pltpu.VMEM),
        ],
        out_specs=pl.BlockSpec(memory_space=pltpu.SMEM),
    )(pre2d, f2d)
    return loss[0, 0]


# speedup vs baseline: 4.0912x; 1.2944x over previous
import jax
import jax.numpy as jnp
from jax.experimental import pallas as pl
from jax.experimental.pallas import tpu as pltpu

GRID_N = 1024
_H = 1.0 / (GRID_N + 1)
_INV_H2 = 1.0 / (_H * _H)
_MAXITER = 5


def _matvec2d(u):
    n = GRID_N
    zrow = jnp.zeros((1, n), jnp.float32)
    zcol = jnp.zeros((n, 1), jnp.float32)
    up = jnp.concatenate([zrow, u[:-1, :]], axis=0)
    down = jnp.concatenate([u[1:, :], zrow], axis=0)
    left = jnp.concatenate([zcol, u[:, :-1]], axis=1)
    right = jnp.concatenate([u[:, 1:], zcol], axis=1)
    lap = (4.0 * u - up - down - left - right) * jnp.float32(_INV_H2)
    return lap + u


def _descent_kernel(pre_ref, f_ref, out_ref):
    n = GRID_N
    row = jax.lax.broadcasted_iota(jnp.int32, (n, n), 0).astype(jnp.float32)
    col = jax.lax.broadcasted_iota(jnp.int32, (n, n), 1).astype(jnp.float32)
    xs_r = (row + 1.0) * jnp.float32(_H)
    xs_c = (col + 1.0) * jnp.float32(_H)
    phi = (xs_r * (1.0 - xs_r)) * (xs_c * (1.0 - xs_c))

    u = pre_ref[...] * phi
    b = f_ref[...]
    r = b - _matvec2d(u)
    diff = jnp.zeros((n, n), jnp.float32)
    for k in range(_MAXITER):
        ar = _matvec2d(r)
        alpha = jnp.sum(r * r) / jnp.sum(r * ar)
        diff = diff + alpha * r
        if k + 1 < _MAXITER:
            r = r - alpha * ar
    out_ref[0, 0] = jnp.sum(diff * diff) / jnp.float32(n * n)


def kernel(pre, f, ans):
    del ans
    pre2d = pre.reshape(GRID_N, GRID_N)
    f2d = f.reshape(GRID_N, GRID_N)
    loss = pl.pallas_call(
        _descent_kernel,
        out_shape=jax.ShapeDtypeStruct((1, 1), jnp.float32),
        in_specs=[
            pl.BlockSpec(memory_space=pltpu.VMEM),
            pl.BlockSpec(memory_space=pltpu.VMEM),
        ],
        out_specs=pl.BlockSpec(memory_space=pltpu.SMEM),
    )(pre2d, f2d)
    return loss[0, 0]


# speedup vs baseline: 4.1483x; 1.0140x over previous
import jax
import jax.numpy as jnp
from jax.experimental import pallas as pl
from jax.experimental.pallas import tpu as pltpu

GRID_N = 1024
_H = 1.0 / (GRID_N + 1)
_INV_H2 = 1.0 / (_H * _H)
_MAXITER = 5


def _matvec2d(u):
    n = GRID_N
    zrow = jnp.zeros((1, n), jnp.float32)
    zcol = jnp.zeros((n, 1), jnp.float32)
    up = jnp.concatenate([zrow, u[:-1, :]], axis=0)
    down = jnp.concatenate([u[1:, :], zrow], axis=0)
    left = jnp.concatenate([zcol, u[:, :-1]], axis=1)
    right = jnp.concatenate([u[:, 1:], zcol], axis=1)
    c0 = jnp.float32(4.0 + _H * _H)
    return (c0 * u - up - down - left - right) * jnp.float32(_INV_H2)


def _descent_kernel(pre_ref, f_ref, out_ref):
    n = GRID_N
    row = jax.lax.broadcasted_iota(jnp.int32, (n, n), 0).astype(jnp.float32)
    col = jax.lax.broadcasted_iota(jnp.int32, (n, n), 1).astype(jnp.float32)
    xs_r = (row + 1.0) * jnp.float32(_H)
    xs_c = (col + 1.0) * jnp.float32(_H)
    phi = (xs_r * (1.0 - xs_r)) * (xs_c * (1.0 - xs_c))

    u = pre_ref[...] * phi
    b = f_ref[...]
    r = b - _matvec2d(u)
    diff = jnp.zeros((n, n), jnp.float32)
    for k in range(_MAXITER):
        ar = _matvec2d(r)
        alpha = jnp.sum(r * r) / jnp.sum(r * ar)
        diff = diff + alpha * r
        if k + 1 < _MAXITER:
            r = r - alpha * ar
    out_ref[0, 0] = jnp.sum(diff * diff) / jnp.float32(n * n)


def kernel(pre, f, ans):
    del ans
    pre2d = pre.reshape(GRID_N, GRID_N)
    f2d = f.reshape(GRID_N, GRID_N)
    loss = pl.pallas_call(
        _descent_kernel,
        out_shape=jax.ShapeDtypeStruct((1, 1), jnp.float32),
        in_specs=[
            pl.BlockSpec(memory_space=pltpu.VMEM),
            pl.BlockSpec(memory_space=pltpu.VMEM),
        ],
        out_specs=pl.BlockSpec(memory_space=pltpu.SMEM),
    )(pre2d, f2d)
    return loss[0, 0]
